# baseline (speedup 1.0000x reference)
import functools

import numpy as np
import jax
import jax.numpy as jnp
from jax.experimental import pallas as pl
from jax.experimental.pallas import tpu as pltpu

F32 = jnp.float32
_gelu = jax.nn.gelu


def _enc_body(x_ref, Wa_ref, ba_ref, Wb_ref, bb_ref, G4_ref, R84_ref, R21_ref,
              out_ref):
    xb = x_ref[...]
    h = _gelu(jnp.dot(xb, Wa_ref[...]) + ba_ref[...])
    a = jnp.dot(R84_ref[...], h)
    g = jnp.dot(G4_ref[...], xb)
    hb = _gelu(jnp.dot(g, Wb_ref[...]) + bb_ref[...])
    b = jnp.dot(R21_ref[...], hb)
    out_ref[:, 0:256] = a
    out_ref[:, 256:512] = b


def _patch_encoder(x2, Wa, ba, Wb, bb):
    G4 = jnp.asarray(np.kron(np.eye(168), np.ones((1, 4)) / 4.0), F32)
    R84 = jnp.asarray(np.kron(np.eye(8), np.ones((1, 84)) / 84.0), F32)
    R21 = jnp.asarray(np.kron(np.eye(8), np.ones((1, 21)) / 21.0), F32)
    n_blk = 32
    return pl.pallas_call(
        _enc_body,
        grid=(n_blk,),
        in_specs=[
            pl.BlockSpec((672, 1024), lambda i: (i, 0)),
            pl.BlockSpec((1024, 256), lambda i: (0, 0)),
            pl.BlockSpec((256,), lambda i: (0,)),
            pl.BlockSpec((1024, 256), lambda i: (0, 0)),
            pl.BlockSpec((256,), lambda i: (0,)),
            pl.BlockSpec((168, 672), lambda i: (0, 0)),
            pl.BlockSpec((8, 672), lambda i: (0, 0)),
            pl.BlockSpec((8, 168), lambda i: (0, 0)),
        ],
        out_specs=pl.BlockSpec((8, 512), lambda i: (i, 0)),
        out_shape=jax.ShapeDtypeStruct((256, 512), F32),
    )(x2, Wa, ba, Wb, bb, G4, R84, R21)


def _head_body(xc_ref, Wf_ref, bf_ref, C_ref, out_ref):
    path = _gelu(jnp.dot(xc_ref[...], Wf_ref[...]) + bf_ref[...])
    sim = jax.lax.dot_general(C_ref[...], path,
                              (((1,), (1,)), ((), ()))) * (1.0 / 16.0)
    A = jax.nn.softmax(sim, axis=1)
    out_ref[...] = jnp.dot(A, path)


def _path_head(xc, Wf, bf, C):
    return pl.pallas_call(
        _head_body,
        out_shape=jax.ShapeDtypeStruct((64, 256), F32),
    )(xc, Wf, bf, C)


def _gen_body(x_ref, G1_ref, g1_ref, G2_ref, g2_ref, G3_ref, g3_ref, out_ref):
    def elu(v):
        return jnp.where(v > 0, v, jnp.exp(v) - 1.0)
    h = elu(jnp.dot(x_ref[0], G1_ref[0]) + g1_ref[0])
    h = elu(jnp.dot(h, G2_ref[0]) + g2_ref[0])
    out_ref[0] = elu(jnp.dot(h, G3_ref[0]) + g3_ref[0])


def _genomics(x_omics, G1, g1, G2, g2, G3, g3):
    out = pl.pallas_call(
        _gen_body,
        grid=(64,),
        in_specs=[
            pl.BlockSpec((1, 1, 300), lambda i: (i, 0, 0)),
            pl.BlockSpec((1, 300, 1024), lambda i: (i, 0, 0)),
            pl.BlockSpec((1, 1, 1024), lambda i: (i, 0, 0)),
            pl.BlockSpec((1, 1024, 512), lambda i: (i, 0, 0)),
            pl.BlockSpec((1, 1, 512), lambda i: (i, 0, 0)),
            pl.BlockSpec((1, 512, 256), lambda i: (i, 0, 0)),
            pl.BlockSpec((1, 1, 256), lambda i: (i, 0, 0)),
        ],
        out_specs=pl.BlockSpec((1, 1, 256), lambda i: (i, 0, 0)),
        out_shape=jax.ShapeDtypeStruct((64, 1, 256), F32),
    )(x_omics.reshape(64, 1, 300), G1, g1.reshape(64, 1, 1024),
      G2, g2.reshape(64, 1, 512), G3, g3.reshape(64, 1, 256))
    return out.reshape(64, 256)


def _route_body(t_ref, cross_ref, Wg_ref, bg_ref, full_ref):
    ctx = jnp.mean(cross_ref[...], axis=0, keepdims=True)
    z = t_ref[...] + ctx
    logits = jnp.dot(z, Wg_ref[...]) + bg_ref[...]
    work = logits
    ohs, vals = [], []
    for _ in range(4):
        m = jnp.max(work, axis=1, keepdims=True)
        oh = work >= m
        ohs.append(oh)
        vals.append(m)
        work = jnp.where(oh, -jnp.inf, work)
    V = jnp.concatenate(vals, axis=1)
    g = jax.nn.softmax(V, axis=1)
    full = jnp.zeros_like(logits)
    for j in range(4):
        full = full + jnp.where(ohs[j], g[:, j:j + 1], 0.0)
    full_ref[...] = full


def _routing(t, cross, Wg, bg):
    return pl.pallas_call(
        _route_body,
        out_shape=jax.ShapeDtypeStruct((64, 64), F32),
    )(t, cross, Wg, bg)


_E_BLK = 8


def _moe_body(t_ref, full_ref, W1_ref, b1_ref, W2_ref, b2_ref, out_ref):
    t = t_ref[...]

    @pl.when(pl.program_id(0) == 0)
    def _():
        out_ref[...] = t

    base = pl.program_id(0) * _E_BLK
    row = jax.lax.broadcasted_iota(jnp.int32, (64, _E_BLK), 0)
    col = jax.lax.broadcasted_iota(jnp.int32, (64, _E_BLK), 1)
    onehot = (row == base + col).astype(F32)
    gates = jnp.dot(full_ref[...], onehot)

    acc = jnp.zeros((64, 256), F32)
    for le in range(_E_BLK):
        h = _gelu(jnp.dot(t, W1_ref[le]) + b1_ref[le])
        eo = jnp.dot(h, W2_ref[le]) + b2_ref[le]
        acc = acc + gates[:, le:le + 1] * eo
    out_ref[...] += acc


def _moe_experts(t, full, W1, b1, W2, b2):
    n_blk = 64 // _E_BLK
    return pl.pallas_call(
        _moe_body,
        grid=(n_blk,),
        in_specs=[
            pl.BlockSpec((64, 256), lambda i: (0, 0)),
            pl.BlockSpec((64, 64), lambda i: (0, 0)),
            pl.BlockSpec((_E_BLK, 256, 512), lambda i: (i, 0, 0)),
            pl.BlockSpec((_E_BLK, 512), lambda i: (i, 0)),
            pl.BlockSpec((_E_BLK, 512, 256), lambda i: (i, 0, 0)),
            pl.BlockSpec((_E_BLK, 256), lambda i: (i, 0)),
        ],
        out_specs=pl.BlockSpec((64, 256), lambda i: (0, 0)),
        out_shape=jax.ShapeDtypeStruct((64, 256), F32),
    )(t, full, W1, b1, W2, b2)


def _mha_pair_body(p_ref, g_ref,
                   pWq, pWk, pWv, pWo, pbq, pbk, pbv, pbo,
                   gWq, gWk, gWv, gWo, gbq, gbk, gbv, gbo,
                   pig_ref, gip_ref):
    path = p_ref[...]
    gen = g_ref[...]

    def mha(q_in, kv_in, Wq, bq, Wk, bk, Wv, bv, Wo, bo):
        Q = jnp.dot(q_in, Wq[...]) + bq[...]
        K = jnp.dot(kv_in, Wk[...]) + bk[...]
        V = jnp.dot(kv_in, Wv[...]) + bv[...]
        att = jax.nn.softmax(
            jax.lax.dot_general(Q, K, (((1,), (1,)), ((), ()))) * (1.0 / 16.0),
            axis=1)
        return jnp.dot(jnp.dot(att, V), Wo[...]) + bo[...]

    pig_ref[...] = mha(path, gen, pWq, pbq, pWk, pbk, pWv, pbv, pWo, pbo)
    gip_ref[...] = mha(gen, path, gWq, gbq, gWk, gbk, gWv, gbv, gWo, gbo)


def _mha_pair(path, gen, pg_w, gp_w):
    return pl.pallas_call(
        _mha_pair_body,
        out_shape=(jax.ShapeDtypeStruct((64, 256), F32),
                   jax.ShapeDtypeStruct((64, 256), F32)),
    )(path, gen, *pg_w, *gp_w)


def _final_body(pig_ref, gip_ref, ppw_ref, ppb_ref, gpw_ref, gpb_ref,
                Wm1_ref, bm1_ref, Wm2_ref, bm2_ref, Wc_ref, bc_ref,
                hz_ref, S_ref):
    pig = pig_ref[...]
    gip = gip_ref[...]

    def sap(x, w, b):
        s = jnp.dot(x, w[...]) + b[...]
        a = jax.nn.softmax(s, axis=0)
        return jax.lax.dot_general(a, x, (((0,), (0,)), ((), ())))

    pf = sap(pig, ppw_ref, ppb_ref)
    gf = sap(gip, gpw_ref, gpb_ref)
    f = jnp.concatenate([pf, gf], axis=1)
    f = _gelu(jnp.dot(f, Wm1_ref[...]) + bm1_ref[...])
    f = _gelu(jnp.dot(f, Wm2_ref[...]) + bm2_ref[...])
    logits = jnp.dot(f, Wc_ref[...]) + bc_ref[...]
    hz = jax.nn.sigmoid(logits)
    one = 1.0 - hz
    c1 = one[:, 0:1]
    c2 = c1 * one[:, 1:2]
    c3 = c2 * one[:, 2:3]
    c4 = c3 * one[:, 3:4]
    hz_ref[...] = hz
    S_ref[...] = jnp.concatenate([c1, c2, c3, c4], axis=1)


def _final(pig, gip, pp_w, pp_b, gpool_w, gpool_b, Wm1, bm1, Wm2, bm2, Wc, bc):
    return pl.pallas_call(
        _final_body,
        out_shape=(jax.ShapeDtypeStruct((1, 4), F32),
                   jax.ShapeDtypeStruct((1, 4), F32)),
    )(pig, gip, pp_w, pp_b, gpool_w, gpool_b, Wm1, bm1, Wm2, bm2, Wc, bc)


def kernel(x_path, x_omics, Wa, ba, Wb, bb, Wf, bf, G1, g1, G2, g2, G3, g3, C,
           pe_Wg, pe_bg, pe_W1, pe_b1, pe_W2, pe_b2,
           ge_Wg, ge_bg, ge_W1, ge_b1, ge_W2, ge_b2,
           pd_Wg, pd_bg, pd_W1, pd_b1, pd_W2, pd_b2,
           gd_Wg, gd_bg, gd_W1, gd_b1, gd_W2, gd_b2,
           pg_Wq, pg_Wk, pg_Wv, pg_Wo, pg_bq, pg_bk, pg_bv, pg_bo,
           gp_Wq, gp_Wk, gp_Wv, gp_Wo, gp_bq, gp_bk, gp_bv, gp_bo,
           pp_w, pp_b, gpool_w, gpool_b, Wm1, bm1, Wm2, bm2, Wc, bc):
    x2 = x_path.reshape(21504, 1024)
    gen = _genomics(x_omics, G1, g1, G2, g2, G3, g3)
    xc = _patch_encoder(x2, Wa, ba, Wb, bb)
    path0 = _path_head(xc, Wf, bf, C)

    pe_full = _routing(path0, gen, pe_Wg, pe_bg)
    path1 = _moe_experts(path0, pe_full, pe_W1, pe_b1, pe_W2, pe_b2)
    ge_full = _routing(gen, path1, ge_Wg, ge_bg)
    gen1 = _moe_experts(gen, ge_full, ge_W1, ge_b1, ge_W2, ge_b2)

    pig, gip = _mha_pair(
        path1, gen1,
        (pg_Wq, pg_Wk, pg_Wv, pg_Wo, pg_bq, pg_bk, pg_bv, pg_bo),
        (gp_Wq, gp_Wk, gp_Wv, gp_Wo, gp_bq, gp_bk, gp_bv, gp_bo))

    pd_full = _routing(pig, gip, pd_Wg, pd_bg)
    pig1 = _moe_experts(pig, pd_full, pd_W1, pd_b1, pd_W2, pd_b2)
    gd_full = _routing(gip, pig1, gd_Wg, gd_bg)
    gip1 = _moe_experts(gip, gd_full, gd_W1, gd_b1, gd_W2, gd_b2)

    hz, S = _final(pig1, gip1, pp_w, pp_b, gpool_w, gpool_b,
                   Wm1, bm1, Wm2, bm2, Wc, bc)
    return hz, S, pe_full, ge_full, pd_full, gd_full


# speedup vs baseline: 1.1717x; 1.1717x over previous
import functools

import numpy as np
import jax
import jax.numpy as jnp
from jax.experimental import pallas as pl
from jax.experimental.pallas import tpu as pltpu

F32 = jnp.float32
_gelu = jax.nn.gelu


_ENC_B = 8


def _enc_body(x_ref, Wa_ref, ba_ref, Wb_ref, bb_ref, G4b_ref, out_ref):
    Wa = Wa_ref[...]
    Wb = Wb_ref[...]
    ba = ba_ref[...]
    bb = bb_ref[...]
    G4b = G4b_ref[...]
    for b in range(_ENC_B):
        xb = x_ref[b]
        h = _gelu(jnp.dot(xb, Wa) + ba)
        a = jnp.mean(h, axis=0)
        g = jnp.dot(G4b, xb)
        hb = _gelu(jnp.dot(g, Wb) + bb)
        bm = jnp.mean(hb, axis=0)
        out_ref[b, 0:256] = a
        out_ref[b, 256:512] = bm


def _patch_encoder(x3, Wa, ba, Wb, bb):
    G4b = jnp.asarray(np.kron(np.eye(21), np.ones((1, 4)) / 4.0), F32)
    n_blk = 256 // _ENC_B
    return pl.pallas_call(
        _enc_body,
        grid=(n_blk,),
        in_specs=[
            pl.BlockSpec((_ENC_B, 84, 1024), lambda i: (i, 0, 0)),
            pl.BlockSpec((1024, 256), lambda i: (0, 0)),
            pl.BlockSpec((256,), lambda i: (0,)),
            pl.BlockSpec((1024, 256), lambda i: (0, 0)),
            pl.BlockSpec((256,), lambda i: (0,)),
            pl.BlockSpec((21, 84), lambda i: (0, 0)),
        ],
        out_specs=pl.BlockSpec((_ENC_B, 512), lambda i: (i, 0)),
        out_shape=jax.ShapeDtypeStruct((256, 512), F32),
    )(x3, Wa, ba, Wb, bb, G4b)


def _head_body(xc_ref, Wf_ref, bf_ref, C_ref, out_ref):
    path = _gelu(jnp.dot(xc_ref[...], Wf_ref[...]) + bf_ref[...])
    sim = jax.lax.dot_general(C_ref[...], path,
                              (((1,), (1,)), ((), ()))) * (1.0 / 16.0)
    A = jax.nn.softmax(sim, axis=1)
    out_ref[...] = jnp.dot(A, path)


def _path_head(xc, Wf, bf, C):
    return pl.pallas_call(
        _head_body,
        out_shape=jax.ShapeDtypeStruct((64, 256), F32),
    )(xc, Wf, bf, C)


def _gen_body(x_ref, G1_ref, g1_ref, G2_ref, g2_ref, G3_ref, g3_ref, out_ref):
    def elu(v):
        return jnp.where(v > 0, v, jnp.exp(v) - 1.0)
    h = elu(jnp.dot(x_ref[0], G1_ref[0]) + g1_ref[0])
    h = elu(jnp.dot(h, G2_ref[0]) + g2_ref[0])
    out_ref[0] = elu(jnp.dot(h, G3_ref[0]) + g3_ref[0])


def _genomics(x_omics, G1, g1, G2, g2, G3, g3):
    out = pl.pallas_call(
        _gen_body,
        grid=(64,),
        in_specs=[
            pl.BlockSpec((1, 1, 300), lambda i: (i, 0, 0)),
            pl.BlockSpec((1, 300, 1024), lambda i: (i, 0, 0)),
            pl.BlockSpec((1, 1, 1024), lambda i: (i, 0, 0)),
            pl.BlockSpec((1, 1024, 512), lambda i: (i, 0, 0)),
            pl.BlockSpec((1, 1, 512), lambda i: (i, 0, 0)),
            pl.BlockSpec((1, 512, 256), lambda i: (i, 0, 0)),
            pl.BlockSpec((1, 1, 256), lambda i: (i, 0, 0)),
        ],
        out_specs=pl.BlockSpec((1, 1, 256), lambda i: (i, 0, 0)),
        out_shape=jax.ShapeDtypeStruct((64, 1, 256), F32),
    )(x_omics.reshape(64, 1, 300), G1, g1.reshape(64, 1, 1024),
      G2, g2.reshape(64, 1, 512), G3, g3.reshape(64, 1, 256))
    return out.reshape(64, 256)


def _route_body(t_ref, cross_ref, Wg_ref, bg_ref, full_ref):
    ctx = jnp.mean(cross_ref[...], axis=0, keepdims=True)
    z = t_ref[...] + ctx
    logits = jnp.dot(z, Wg_ref[...]) + bg_ref[...]
    work = logits
    ohs, vals = [], []
    for _ in range(4):
        m = jnp.max(work, axis=1, keepdims=True)
        oh = work >= m
        ohs.append(oh)
        vals.append(m)
        work = jnp.where(oh, -jnp.inf, work)
    V = jnp.concatenate(vals, axis=1)
    g = jax.nn.softmax(V, axis=1)
    full = jnp.zeros_like(logits)
    for j in range(4):
        full = full + jnp.where(ohs[j], g[:, j:j + 1], 0.0)
    full_ref[...] = full


def _routing(t, cross, Wg, bg):
    return pl.pallas_call(
        _route_body,
        out_shape=jax.ShapeDtypeStruct((64, 64), F32),
    )(t, cross, Wg, bg)


_E_BLK = 8


def _moe_body(t_ref, full_ref, W1_ref, b1_ref, W2_ref, b2_ref, out_ref):
    t = t_ref[...]

    @pl.when(pl.program_id(0) == 0)
    def _():
        out_ref[...] = t

    base = pl.program_id(0) * _E_BLK
    row = jax.lax.broadcasted_iota(jnp.int32, (64, _E_BLK), 0)
    col = jax.lax.broadcasted_iota(jnp.int32, (64, _E_BLK), 1)
    onehot = (row == base + col).astype(F32)
    gates = jnp.dot(full_ref[...], onehot)

    acc = jnp.zeros((64, 256), F32)
    for le in range(_E_BLK):
        h = _gelu(jnp.dot(t, W1_ref[le]) + b1_ref[le])
        eo = jnp.dot(h, W2_ref[le]) + b2_ref[le]
        acc = acc + gates[:, le:le + 1] * eo
    out_ref[...] += acc


def _moe_experts(t, full, W1, b1, W2, b2):
    n_blk = 64 // _E_BLK
    return pl.pallas_call(
        _moe_body,
        grid=(n_blk,),
        in_specs=[
            pl.BlockSpec((64, 256), lambda i: (0, 0)),
            pl.BlockSpec((64, 64), lambda i: (0, 0)),
            pl.BlockSpec((_E_BLK, 256, 512), lambda i: (i, 0, 0)),
            pl.BlockSpec((_E_BLK, 512), lambda i: (i, 0)),
            pl.BlockSpec((_E_BLK, 512, 256), lambda i: (i, 0, 0)),
            pl.BlockSpec((_E_BLK, 256), lambda i: (i, 0)),
        ],
        out_specs=pl.BlockSpec((64, 256), lambda i: (0, 0)),
        out_shape=jax.ShapeDtypeStruct((64, 256), F32),
    )(t, full, W1, b1, W2, b2)


def _mha_pair_body(p_ref, g_ref,
                   pWq, pWk, pWv, pWo, pbq, pbk, pbv, pbo,
                   gWq, gWk, gWv, gWo, gbq, gbk, gbv, gbo,
                   pig_ref, gip_ref):
    path = p_ref[...]
    gen = g_ref[...]

    def mha(q_in, kv_in, Wq, bq, Wk, bk, Wv, bv, Wo, bo):
        Q = jnp.dot(q_in, Wq[...]) + bq[...]
        K = jnp.dot(kv_in, Wk[...]) + bk[...]
        V = jnp.dot(kv_in, Wv[...]) + bv[...]
        att = jax.nn.softmax(
            jax.lax.dot_general(Q, K, (((1,), (1,)), ((), ()))) * (1.0 / 16.0),
            axis=1)
        return jnp.dot(jnp.dot(att, V), Wo[...]) + bo[...]

    pig_ref[...] = mha(path, gen, pWq, pbq, pWk, pbk, pWv, pbv, pWo, pbo)
    gip_ref[...] = mha(gen, path, gWq, gbq, gWk, gbk, gWv, gbv, gWo, gbo)


def _mha_pair(path, gen, pg_w, gp_w):
    return pl.pallas_call(
        _mha_pair_body,
        out_shape=(jax.ShapeDtypeStruct((64, 256), F32),
                   jax.ShapeDtypeStruct((64, 256), F32)),
    )(path, gen, *pg_w, *gp_w)


def _final_body(pig_ref, gip_ref, ppw_ref, ppb_ref, gpw_ref, gpb_ref,
                Wm1_ref, bm1_ref, Wm2_ref, bm2_ref, Wc_ref, bc_ref,
                hz_ref, S_ref):
    pig = pig_ref[...]
    gip = gip_ref[...]

    def sap(x, w, b):
        s = jnp.dot(x, w[...]) + b[...]
        a = jax.nn.softmax(s, axis=0)
        return jax.lax.dot_general(a, x, (((0,), (0,)), ((), ())))

    pf = sap(pig, ppw_ref, ppb_ref)
    gf = sap(gip, gpw_ref, gpb_ref)
    f = jnp.concatenate([pf, gf], axis=1)
    f = _gelu(jnp.dot(f, Wm1_ref[...]) + bm1_ref[...])
    f = _gelu(jnp.dot(f, Wm2_ref[...]) + bm2_ref[...])
    logits = jnp.dot(f, Wc_ref[...]) + bc_ref[...]
    hz = jax.nn.sigmoid(logits)
    one = 1.0 - hz
    c1 = one[:, 0:1]
    c2 = c1 * one[:, 1:2]
    c3 = c2 * one[:, 2:3]
    c4 = c3 * one[:, 3:4]
    hz_ref[...] = hz
    S_ref[...] = jnp.concatenate([c1, c2, c3, c4], axis=1)


def _final(pig, gip, pp_w, pp_b, gpool_w, gpool_b, Wm1, bm1, Wm2, bm2, Wc, bc):
    return pl.pallas_call(
        _final_body,
        out_shape=(jax.ShapeDtypeStruct((1, 4), F32),
                   jax.ShapeDtypeStruct((1, 4), F32)),
    )(pig, gip, pp_w, pp_b, gpool_w, gpool_b, Wm1, bm1, Wm2, bm2, Wc, bc)


def kernel(x_path, x_omics, Wa, ba, Wb, bb, Wf, bf, G1, g1, G2, g2, G3, g3, C,
           pe_Wg, pe_bg, pe_W1, pe_b1, pe_W2, pe_b2,
           ge_Wg, ge_bg, ge_W1, ge_b1, ge_W2, ge_b2,
           pd_Wg, pd_bg, pd_W1, pd_b1, pd_W2, pd_b2,
           gd_Wg, gd_bg, gd_W1, gd_b1, gd_W2, gd_b2,
           pg_Wq, pg_Wk, pg_Wv, pg_Wo, pg_bq, pg_bk, pg_bv, pg_bo,
           gp_Wq, gp_Wk, gp_Wv, gp_Wo, gp_bq, gp_bk, gp_bv, gp_bo,
           pp_w, pp_b, gpool_w, gpool_b, Wm1, bm1, Wm2, bm2, Wc, bc):
    gen = _genomics(x_omics, G1, g1, G2, g2, G3, g3)
    xc = _patch_encoder(x_path[0], Wa, ba, Wb, bb)
    path0 = _path_head(xc, Wf, bf, C)

    pe_full = _routing(path0, gen, pe_Wg, pe_bg)
    path1 = _moe_experts(path0, pe_full, pe_W1, pe_b1, pe_W2, pe_b2)
    ge_full = _routing(gen, path1, ge_Wg, ge_bg)
    gen1 = _moe_experts(gen, ge_full, ge_W1, ge_b1, ge_W2, ge_b2)

    pig, gip = _mha_pair(
        path1, gen1,
        (pg_Wq, pg_Wk, pg_Wv, pg_Wo, pg_bq, pg_bk, pg_bv, pg_bo),
        (gp_Wq, gp_Wk, gp_Wv, gp_Wo, gp_bq, gp_bk, gp_bv, gp_bo))

    pd_full = _routing(pig, gip, pd_Wg, pd_bg)
    pig1 = _moe_experts(pig, pd_full, pd_W1, pd_b1, pd_W2, pd_b2)
    gd_full = _routing(gip, pig1, gd_Wg, gd_bg)
    gip1 = _moe_experts(gip, gd_full, gd_W1, gd_b1, gd_W2, gd_b2)

    hz, S = _final(pig1, gip1, pp_w, pp_b, gpool_w, gpool_b,
                   Wm1, bm1, Wm2, bm2, Wc, bc)
    return hz, S, pe_full, ge_full, pd_full, gd_full
